# baseline (device time: 34340 ns/iter reference)
import jax
import jax.numpy as jnp
from jax import lax
from jax.experimental import pallas as pl
from jax.experimental.pallas import tpu as pltpu

N_DEV = 16
E_PER = 2
N_EXP = 32
CAP = 204
T_LOC = 512
D = 256
H = 512
SEG = 64
FLAT = N_DEV * SEG


def kernel(x, router_W, route_idx, expert_W):
    del router_W

    def body(x_ref, ridx_ref, ew_ref, out_ref,
             sbuf, rbuf, ybuf, yret, scbuf, scret, cnt_ref,
             dsend, drecv, ssend, srecv, rsend, rrecv, qsend, qrecv):
        my = lax.axis_index("i")

        r = ridx_ref[...]
        eids = lax.broadcasted_iota(jnp.int32, (T_LOC, N_EXP), 1)
        onehot = (r == eids).astype(jnp.float32)
        totals = jnp.sum(onehot, axis=0, keepdims=True)
        cnt_ref[pl.ds(my, 1), :] = totals

        creqs = []
        for k in range(1, N_DEV):
            tgt = lax.rem(my + k, N_DEV)
            cr = pltpu.make_async_remote_copy(
                src_ref=cnt_ref.at[pl.ds(my, 1)],
                dst_ref=cnt_ref.at[pl.ds(my, 1)],
                send_sem=dsend.at[k - 1],
                recv_sem=drecv.at[k - 1],
                device_id=(tgt,),
                device_id_type=pl.DeviceIdType.MESH,
            )
            cr.start()
            creqs.append(cr)

        ti = lax.broadcasted_iota(jnp.int32, (T_LOC, T_LOC), 0)
        tj = lax.broadcasted_iota(jnp.int32, (T_LOC, T_LOC), 1)
        tril = (tj < ti).astype(jnp.float32)
        excl = jnp.dot(tril, onehot, preferred_element_type=jnp.float32)

        rankE = jnp.sum(onehot * excl, axis=1, keepdims=True)
        rankE_i = rankE.astype(jnp.int32)
        dev = lax.div(r, E_PER)
        j_rel = lax.rem(dev - my + N_DEV, N_DEV)
        kk_t = lax.rem(r, E_PER)

        lane = lax.broadcasted_iota(jnp.int32, (1, N_EXP), 1)
        tshift = jnp.concatenate(
            [jnp.zeros((1, 1), jnp.float32), totals[:, :N_EXP - 1]], axis=1)
        c0_tok = jnp.sum(onehot * tshift, axis=1, keepdims=True)
        off = jnp.where(kk_t == 1, c0_tok.astype(jnp.int32), 0)

        in_seg = off + rankE_i
        slot = j_rel * SEG + in_seg
        slot = jnp.where(in_seg < SEG, slot, -1)
        sl_ids = lax.broadcasted_iota(jnp.int32, (T_LOC, FLAT), 1)
        perm = (slot == sl_ids).astype(jnp.bfloat16)

        xbf = x_ref[...].astype(jnp.bfloat16)
        packed = lax.dot_general(
            perm, xbf, (((0,), (0,)), ((), ())),
            preferred_element_type=jnp.float32)
        sbuf[...] = packed.astype(jnp.bfloat16).reshape(N_DEV, SEG, D)

        dreqs = []
        for k in range(1, N_DEV):
            tgt = lax.rem(my + k, N_DEV)
            dr = pltpu.make_async_remote_copy(
                src_ref=sbuf.at[k],
                dst_ref=rbuf.at[N_DEV - k],
                send_sem=ssend.at[k - 1],
                recv_sem=srecv.at[k - 1],
                device_id=(tgt,),
                device_id_type=pl.DeviceIdType.MESH,
            )
            dr.start()
            dreqs.append(dr)
        rbuf[0] = sbuf[0]

        wbf = ew_ref[...].astype(jnp.bfloat16)

        def compute_slots(lo, hi, m0):
            n = hi - lo
            rows = rbuf[lo:hi, :, :].reshape(n * SEG, D)
            y = jnp.dot(rows * m0, wbf[0], preferred_element_type=jnp.float32)
            y += jnp.dot(rows * (1.0 - m0).astype(jnp.bfloat16), wbf[1],
                         preferred_element_type=jnp.float32)
            y3 = y.reshape(n, SEG, H)
            m3 = jnp.maximum(jnp.max(jnp.abs(y3), axis=2, keepdims=True), 1e-20)
            ybuf[lo:hi, :, :] = jnp.rint(y3 * (127.0 / m3)).astype(jnp.int8)
            scbuf[lo:hi, :, :] = m3 * (1.0 / 127.0)

        def start_return(j):
            tgt = lax.rem(my + j, N_DEV)
            rr = pltpu.make_async_remote_copy(
                src_ref=ybuf.at[j],
                dst_ref=yret.at[N_DEV - j],
                send_sem=rsend.at[j - 1],
                recv_sem=rrecv.at[j - 1],
                device_id=(tgt,),
                device_id_type=pl.DeviceIdType.MESH,
            )
            rr.start()
            qq = pltpu.make_async_remote_copy(
                src_ref=scbuf.at[j],
                dst_ref=scret.at[N_DEV - j],
                send_sem=qsend.at[j - 1],
                recv_sem=qrecv.at[j - 1],
                device_id=(tgt,),
                device_id_type=pl.DeviceIdType.MESH,
            )
            qq.start()
            return rr, qq

        within1 = lax.broadcasted_iota(jnp.int32, (SEG, 1), 0)
        my_col = (lane == E_PER * my).astype(jnp.float32)
        c0_own = jnp.sum(totals * my_col)
        m0_own = (within1.astype(jnp.float32) < c0_own).astype(jnp.bfloat16)
        compute_slots(0, 1, m0_own)
        yret[0] = ybuf[0]
        scret[0] = scbuf[0]

        for cr in creqs:
            cr.wait_recv()
        c0_vec = jnp.sum(cnt_ref[...] * my_col, axis=1, keepdims=True)
        rowid = lax.broadcasted_iota(jnp.int32, (FLAT, 1), 0)
        seg_dev = lax.rem(lax.div(rowid, SEG) + my, N_DEV)
        within = lax.rem(rowid, SEG).astype(jnp.float32)
        seg_1h = (seg_dev == lax.broadcasted_iota(
            jnp.int32, (FLAT, N_DEV), 1)).astype(jnp.float32)
        c0_row = jnp.dot(seg_1h, c0_vec, preferred_element_type=jnp.float32)
        m0_all = (within < c0_row).astype(jnp.bfloat16)

        rreqs = {}
        for lo in (1, 6, 11):
            hi = lo + 5
            for m in range(lo, hi):
                dreqs[(N_DEV - m) - 1].wait_recv()
            compute_slots(lo, hi, m0_all[lo * SEG:hi * SEG])
            for m in range(lo, hi):
                rreqs[m] = start_return(m)

        def unpack(lo, hi):
            ydq = (yret[lo:hi, :, :].astype(jnp.float32)
                   * scret[lo:hi, :, :]).astype(jnp.bfloat16)
            yflat = ydq.reshape((hi - lo) * SEG, H)
            return jnp.dot(perm[:, lo * SEG:hi * SEG], yflat,
                           preferred_element_type=jnp.float32)

        out = None
        for lo in (12, 8, 4, 0):
            hi = lo + 4
            for s in range(max(lo, 1), hi):
                rreqs[N_DEV - s][0].wait_recv()
                rreqs[N_DEV - s][1].wait_recv()
            part = unpack(lo, hi)
            out = part if out is None else out + part

        rows_i = lax.broadcasted_iota(jnp.int32, (N_DEV, N_EXP), 0)
        rowmask = (rows_i < my).astype(jnp.float32)
        offs = jnp.sum(cnt_ref[...] * rowmask, axis=0, keepdims=True)
        grank = jnp.sum(onehot * (excl + offs), axis=1, keepdims=True)
        keep = (grank < CAP).astype(jnp.float32)
        out_ref[...] = out * keep

        for q in dreqs:
            q.wait_send()
        for rr, qq in rreqs.values():
            rr.wait_send()
            qq.wait_send()
        for cr in creqs:
            cr.wait_send()

    return pl.pallas_call(
        body,
        out_shape=jax.ShapeDtypeStruct((T_LOC, H), jnp.float32),
        in_specs=[pl.BlockSpec(memory_space=pltpu.VMEM)] * 3,
        out_specs=pl.BlockSpec(memory_space=pltpu.VMEM),
        scratch_shapes=[
            pltpu.VMEM((N_DEV, SEG, D), jnp.bfloat16),
            pltpu.VMEM((N_DEV, SEG, D), jnp.bfloat16),
            pltpu.VMEM((N_DEV, SEG, H), jnp.int8),
            pltpu.VMEM((N_DEV, SEG, H), jnp.int8),
            pltpu.VMEM((N_DEV, SEG, 1), jnp.float32),
            pltpu.VMEM((N_DEV, SEG, 1), jnp.float32),
            pltpu.VMEM((N_DEV, N_EXP), jnp.float32),
            pltpu.SemaphoreType.DMA((N_DEV - 1,)),
            pltpu.SemaphoreType.DMA((N_DEV - 1,)),
            pltpu.SemaphoreType.DMA((N_DEV - 1,)),
            pltpu.SemaphoreType.DMA((N_DEV - 1,)),
            pltpu.SemaphoreType.DMA((N_DEV - 1,)),
            pltpu.SemaphoreType.DMA((N_DEV - 1,)),
            pltpu.SemaphoreType.DMA((N_DEV - 1,)),
            pltpu.SemaphoreType.DMA((N_DEV - 1,)),
        ],
    )(x, route_idx, expert_W)


# device time: 34322 ns/iter; 1.0005x vs baseline; 1.0005x over previous
import jax
import jax.numpy as jnp
from jax import lax
from jax.experimental import pallas as pl
from jax.experimental.pallas import tpu as pltpu

N_DEV = 16
E_PER = 2
N_EXP = 32
CAP = 204
T_LOC = 512
D = 256
H = 512
SEG = 64
FLAT = N_DEV * SEG


def kernel(x, router_W, route_idx, expert_W):
    del router_W

    def body(x_ref, ridx_ref, ew_ref, out_ref,
             sbuf, rbuf, ybuf, yret,
             ssend, srecv, rsend, rrecv):
        my = lax.axis_index("i")

        r = ridx_ref[...]
        eids = lax.broadcasted_iota(jnp.int32, (T_LOC, N_EXP), 1)
        onehot = (r == eids).astype(jnp.float32)
        totals = jnp.sum(onehot, axis=0, keepdims=True)

        ti = lax.broadcasted_iota(jnp.int32, (T_LOC, T_LOC), 0)
        tj = lax.broadcasted_iota(jnp.int32, (T_LOC, T_LOC), 1)
        tril = (tj < ti).astype(jnp.float32)
        excl = jnp.dot(tril, onehot, preferred_element_type=jnp.float32)

        rankE = jnp.sum(onehot * excl, axis=1, keepdims=True)
        rankE_i = rankE.astype(jnp.int32)
        dev = lax.div(r, E_PER)
        j_rel = lax.rem(dev - my + N_DEV, N_DEV)
        kk_t = lax.rem(r, E_PER)

        lane = lax.broadcasted_iota(jnp.int32, (1, N_EXP), 1)
        tshift = jnp.concatenate(
            [jnp.zeros((1, 1), jnp.float32), totals[:, :N_EXP - 1]], axis=1)
        c0_tok = jnp.sum(onehot * tshift, axis=1, keepdims=True)
        off = jnp.where(kk_t == 1, c0_tok.astype(jnp.int32), 0)

        in_seg = off + rankE_i
        slot = j_rel * SEG + in_seg
        slot = jnp.where(in_seg < SEG, slot, -1)
        sl_ids = lax.broadcasted_iota(jnp.int32, (T_LOC, FLAT), 1)
        perm = (slot == sl_ids).astype(jnp.bfloat16)

        xbf = x_ref[...].astype(jnp.bfloat16)
        packed = lax.dot_general(
            perm, xbf, (((0,), (0,)), ((), ())),
            preferred_element_type=jnp.float32)
        sbuf[:, :SEG, :] = packed.astype(jnp.bfloat16).reshape(N_DEV, SEG, D)
        t_hi = jnp.floor(totals * (1.0 / 128.0))
        t_lo = totals - 128.0 * t_hi
        enc = jnp.concatenate(
            [t_hi, t_lo, jnp.zeros((1, D - 2 * N_EXP), jnp.float32)],
            axis=1).astype(jnp.bfloat16)
        sbuf[:, SEG:SEG + 1, :] = jnp.broadcast_to(
            enc.reshape(1, 1, D), (N_DEV, 1, D))

        dreqs = []
        for k in range(1, N_DEV):
            tgt = lax.rem(my + k, N_DEV)
            dr = pltpu.make_async_remote_copy(
                src_ref=sbuf.at[k],
                dst_ref=rbuf.at[N_DEV - k],
                send_sem=ssend.at[k - 1],
                recv_sem=srecv.at[k - 1],
                device_id=(tgt,),
                device_id_type=pl.DeviceIdType.MESH,
            )
            dr.start()
            dreqs.append(dr)
        rbuf[0] = sbuf[0]

        wbf = ew_ref[...].astype(jnp.bfloat16)

        def compute_slots(lo, hi, m0):
            n = hi - lo
            rows = rbuf[lo:hi, :SEG, :].reshape(n * SEG, D)
            y = jnp.dot(rows * m0, wbf[0], preferred_element_type=jnp.float32)
            y += jnp.dot(rows * (1.0 - m0).astype(jnp.bfloat16), wbf[1],
                         preferred_element_type=jnp.float32)
            ybuf[lo:hi, :, :] = y.astype(jnp.bfloat16).reshape(n, SEG, H)

        def start_return(j):
            tgt = lax.rem(my + j, N_DEV)
            rr = pltpu.make_async_remote_copy(
                src_ref=ybuf.at[j],
                dst_ref=yret.at[N_DEV - j],
                send_sem=rsend.at[j - 1],
                recv_sem=rrecv.at[j - 1],
                device_id=(tgt,),
                device_id_type=pl.DeviceIdType.MESH,
            )
            rr.start()
            return rr

        within1 = lax.broadcasted_iota(jnp.int32, (SEG, 1), 0)
        my_col = (lane == E_PER * my).astype(jnp.float32)
        c0_own = jnp.sum(totals * my_col)
        m0_own = (within1.astype(jnp.float32) < c0_own).astype(jnp.bfloat16)
        compute_slots(0, 1, m0_own)
        yret[0] = ybuf[0]

        def decode_cnt(lo, hi):
            er = rbuf[lo:hi, SEG, :]
            return (er[:, :N_EXP].astype(jnp.float32) * 128.0
                    + er[:, N_EXP:2 * N_EXP].astype(jnp.float32))

        def group_mask(lo, hi):
            n = hi - lo
            c0g = jnp.sum(decode_cnt(lo, hi) * my_col, axis=1,
                          keepdims=True)
            rowid = lax.broadcasted_iota(jnp.int32, (n * SEG, 1), 0)
            within = lax.rem(rowid, SEG).astype(jnp.float32)
            seg_1h = (lax.div(rowid, SEG) == lax.broadcasted_iota(
                jnp.int32, (n * SEG, n), 1)).astype(jnp.float32)
            c0_row = jnp.dot(seg_1h, c0g,
                             preferred_element_type=jnp.float32)
            return (within < c0_row).astype(jnp.bfloat16)

        rreqs = {}
        for lo in (1, 6, 11):
            hi = lo + 5
            for m in range(lo, hi):
                dreqs[(N_DEV - m) - 1].wait_recv()
            compute_slots(lo, hi, group_mask(lo, hi))
            for m in range(lo, hi):
                rreqs[m] = start_return(m)

        def unpack(lo, hi):
            yflat = yret[lo:hi, :, :].reshape((hi - lo) * SEG, H)
            return jnp.dot(perm[:, lo * SEG:hi * SEG], yflat,
                           preferred_element_type=jnp.float32)

        out = None
        for lo in (12, 8, 4, 0):
            hi = lo + 4
            for s in range(max(lo, 1), hi):
                rreqs[N_DEV - s].wait_recv()
            part = unpack(lo, hi)
            out = part if out is None else out + part

        cnt_slot = decode_cnt(0, N_DEV)
        dev_of = lax.rem(
            lax.broadcasted_iota(jnp.int32, (N_DEV, 1), 0) + my, N_DEV)
        rowmask = (dev_of < my).astype(jnp.float32)
        offs = jnp.sum(cnt_slot * rowmask, axis=0, keepdims=True)
        grank = jnp.sum(onehot * (excl + offs), axis=1, keepdims=True)
        keep = (grank < CAP).astype(jnp.float32)
        out_ref[...] = out * keep

        for q in dreqs:
            q.wait_send()
        for q in rreqs.values():
            q.wait_send()

    return pl.pallas_call(
        body,
        out_shape=jax.ShapeDtypeStruct((T_LOC, H), jnp.float32),
        in_specs=[pl.BlockSpec(memory_space=pltpu.VMEM)] * 3,
        out_specs=pl.BlockSpec(memory_space=pltpu.VMEM),
        scratch_shapes=[
            pltpu.VMEM((N_DEV, SEG + 1, D), jnp.bfloat16),
            pltpu.VMEM((N_DEV, SEG + 1, D), jnp.bfloat16),
            pltpu.VMEM((N_DEV, SEG, H), jnp.bfloat16),
            pltpu.VMEM((N_DEV, SEG, H), jnp.bfloat16),
            pltpu.SemaphoreType.DMA((N_DEV - 1,)),
            pltpu.SemaphoreType.DMA((N_DEV - 1,)),
            pltpu.SemaphoreType.DMA((N_DEV - 1,)),
            pltpu.SemaphoreType.DMA((N_DEV - 1,)),
        ],
    )(x, route_idx, expert_W)


# device time: 29025 ns/iter; 1.1831x vs baseline; 1.1825x over previous
import jax
import jax.numpy as jnp
from jax import lax
from jax.experimental import pallas as pl
from jax.experimental.pallas import tpu as pltpu

N_DEV = 16
E_PER = 2
N_EXP = 32
CAP = 204
T_LOC = 512
D = 256
H = 512
SEG = 64
FLAT = N_DEV * SEG


def kernel(x, router_W, route_idx, expert_W):
    del router_W

    def body(x_ref, ridx_ref, ew_ref, out_ref,
             sbuf, rbuf, ybuf, yret,
             ssend, srecv, rsend, rrecv):
        my = lax.axis_index("i")

        bar = pltpu.get_barrier_semaphore()
        for k in range(1, N_DEV):
            pl.semaphore_signal(
                bar, inc=1,
                device_id=(lax.rem(my + k, N_DEV),),
                device_id_type=pl.DeviceIdType.MESH,
            )

        r = ridx_ref[...]
        eids = lax.broadcasted_iota(jnp.int32, (T_LOC, N_EXP), 1)
        onehot = (r == eids).astype(jnp.float32)
        totals = jnp.sum(onehot, axis=0, keepdims=True)

        ti = lax.broadcasted_iota(jnp.int32, (T_LOC, T_LOC), 0)
        tj = lax.broadcasted_iota(jnp.int32, (T_LOC, T_LOC), 1)
        tril = (tj < ti).astype(jnp.float32)
        excl = jnp.dot(tril, onehot, preferred_element_type=jnp.float32)

        rankE = jnp.sum(onehot * excl, axis=1, keepdims=True)
        rankE_i = rankE.astype(jnp.int32)
        dev = lax.div(r, E_PER)
        j_rel = lax.rem(dev - my + N_DEV, N_DEV)
        kk_t = lax.rem(r, E_PER)

        lane = lax.broadcasted_iota(jnp.int32, (1, N_EXP), 1)
        tshift = jnp.concatenate(
            [jnp.zeros((1, 1), jnp.float32), totals[:, :N_EXP - 1]], axis=1)
        c0_tok = jnp.sum(onehot * tshift, axis=1, keepdims=True)
        off = jnp.where(kk_t == 1, c0_tok.astype(jnp.int32), 0)

        in_seg = off + rankE_i
        slot = j_rel * SEG + in_seg
        slot = jnp.where(in_seg < SEG, slot, -1)
        sl_ids = lax.broadcasted_iota(jnp.int32, (T_LOC, FLAT), 1)
        perm = (slot == sl_ids).astype(jnp.bfloat16)

        xbf = x_ref[...].astype(jnp.bfloat16)
        packed = lax.dot_general(
            perm, xbf, (((0,), (0,)), ((), ())),
            preferred_element_type=jnp.float32)
        sbuf[:, :SEG, :] = packed.astype(jnp.bfloat16).reshape(N_DEV, SEG, D)
        t_hi = jnp.floor(totals * (1.0 / 128.0))
        t_lo = totals - 128.0 * t_hi
        enc = jnp.concatenate(
            [t_hi, t_lo, jnp.zeros((1, D - 2 * N_EXP), jnp.float32)],
            axis=1).astype(jnp.bfloat16)
        sbuf[:, SEG:SEG + 1, :] = jnp.broadcast_to(
            enc.reshape(1, 1, D), (N_DEV, 1, D))

        pl.semaphore_wait(bar, N_DEV - 1)
        dreqs = []
        for k in range(1, N_DEV):
            tgt = lax.rem(my + k, N_DEV)
            dr = pltpu.make_async_remote_copy(
                src_ref=sbuf.at[k],
                dst_ref=rbuf.at[N_DEV - k],
                send_sem=ssend.at[k - 1],
                recv_sem=srecv.at[k - 1],
                device_id=(tgt,),
                device_id_type=pl.DeviceIdType.MESH,
            )
            dr.start()
            dreqs.append(dr)
        rbuf[0] = sbuf[0]

        wbf = ew_ref[...].astype(jnp.bfloat16)

        def compute_slots(lo, hi, m0):
            n = hi - lo
            rows = rbuf[lo:hi, :SEG, :].reshape(n * SEG, D)
            y = jnp.dot(rows * m0, wbf[0], preferred_element_type=jnp.float32)
            y += jnp.dot(rows * (1.0 - m0).astype(jnp.bfloat16), wbf[1],
                         preferred_element_type=jnp.float32)
            ybuf[lo:hi, :, :] = y.astype(jnp.bfloat16).reshape(n, SEG, H)

        def start_return(j):
            tgt = lax.rem(my + j, N_DEV)
            rr = pltpu.make_async_remote_copy(
                src_ref=ybuf.at[j],
                dst_ref=yret.at[N_DEV - j],
                send_sem=rsend.at[j - 1],
                recv_sem=rrecv.at[j - 1],
                device_id=(tgt,),
                device_id_type=pl.DeviceIdType.MESH,
            )
            rr.start()
            return rr

        within1 = lax.broadcasted_iota(jnp.int32, (SEG, 1), 0)
        my_col = (lane == E_PER * my).astype(jnp.float32)
        c0_own = jnp.sum(totals * my_col)
        m0_own = (within1.astype(jnp.float32) < c0_own).astype(jnp.bfloat16)
        compute_slots(0, 1, m0_own)
        yret[0] = ybuf[0]

        def decode_cnt(lo, hi):
            er = rbuf[lo:hi, SEG, :]
            return (er[:, :N_EXP].astype(jnp.float32) * 128.0
                    + er[:, N_EXP:2 * N_EXP].astype(jnp.float32))

        def group_mask(lo, hi):
            n = hi - lo
            c0g = jnp.sum(decode_cnt(lo, hi) * my_col, axis=1,
                          keepdims=True)
            rowid = lax.broadcasted_iota(jnp.int32, (n * SEG, 1), 0)
            within = lax.rem(rowid, SEG).astype(jnp.float32)
            seg_1h = (lax.div(rowid, SEG) == lax.broadcasted_iota(
                jnp.int32, (n * SEG, n), 1)).astype(jnp.float32)
            c0_row = jnp.dot(seg_1h, c0g,
                             preferred_element_type=jnp.float32)
            return (within < c0_row).astype(jnp.bfloat16)

        rreqs = {}
        for lo in (1, 6, 11):
            hi = lo + 5
            for m in range(lo, hi):
                dreqs[(N_DEV - m) - 1].wait_recv()
            compute_slots(lo, hi, group_mask(lo, hi))
            for m in range(lo, hi):
                rreqs[m] = start_return(m)

        def unpack(lo, hi):
            yflat = yret[lo:hi, :, :].reshape((hi - lo) * SEG, H)
            return jnp.dot(perm[:, lo * SEG:hi * SEG], yflat,
                           preferred_element_type=jnp.float32)

        out = None
        for lo in (12, 8, 4, 0):
            hi = lo + 4
            for s in range(max(lo, 1), hi):
                rreqs[N_DEV - s].wait_recv()
            part = unpack(lo, hi)
            out = part if out is None else out + part

        cnt_slot = decode_cnt(0, N_DEV)
        dev_of = lax.rem(
            lax.broadcasted_iota(jnp.int32, (N_DEV, 1), 0) + my, N_DEV)
        rowmask = (dev_of < my).astype(jnp.float32)
        offs = jnp.sum(cnt_slot * rowmask, axis=0, keepdims=True)
        grank = jnp.sum(onehot * (excl + offs), axis=1, keepdims=True)
        keep = (grank < CAP).astype(jnp.float32)
        out_ref[...] = out * keep

        for q in dreqs:
            q.wait_send()
        for q in rreqs.values():
            q.wait_send()

    return pl.pallas_call(
        body,
        out_shape=jax.ShapeDtypeStruct((T_LOC, H), jnp.float32),
        in_specs=[pl.BlockSpec(memory_space=pltpu.VMEM)] * 3,
        out_specs=pl.BlockSpec(memory_space=pltpu.VMEM),
        scratch_shapes=[
            pltpu.VMEM((N_DEV, SEG + 1, D), jnp.bfloat16),
            pltpu.VMEM((N_DEV, SEG + 1, D), jnp.bfloat16),
            pltpu.VMEM((N_DEV, SEG, H), jnp.bfloat16),
            pltpu.VMEM((N_DEV, SEG, H), jnp.bfloat16),
            pltpu.SemaphoreType.DMA((N_DEV - 1,)),
            pltpu.SemaphoreType.DMA((N_DEV - 1,)),
            pltpu.SemaphoreType.DMA((N_DEV - 1,)),
            pltpu.SemaphoreType.DMA((N_DEV - 1,)),
        ],
        compiler_params=pltpu.CompilerParams(collective_id=0),
    )(x, route_idx, expert_W)
